# baseline (speedup 1.0000x reference)
import functools

import jax
import jax.numpy as jnp
from jax import lax
from jax.experimental import pallas as pl
from jax.experimental.pallas import tpu as pltpu
from jax.experimental.pallas import tpu_sc as plsc

NC, NS = 2, 16
NW = NC * NS
NB = 8
RCH = 400
DP = 128


def _mesh():
    return plsc.VectorSubcoreMesh(
        core_axis_name="c", subcore_axis_name="s", num_cores=NC, num_subcores=NS
    )


def _repack(table):
    v, d = table.shape
    nch = v // RCH

    @functools.partial(
        pl.kernel,
        out_type=jax.ShapeDtypeStruct((v, DP), jnp.float32),
        mesh=_mesh(),
        scratch_types=[
            pltpu.VMEM((RCH, d), jnp.float32),
            pltpu.VMEM((RCH, DP), jnp.float32),
        ],
        compiler_params=pltpu.CompilerParams(use_tc_tiling_on_sc=True),
    )
    def run(table_hbm, padded_hbm, tv, tv128):
        cid = lax.axis_index("c")
        sid = lax.axis_index("s")
        wid = sid * NC + cid

        @pl.loop(0, (nch + NW - 1) // NW)
        def _(i):
            ch = wid + i * NW

            @pl.when(ch < nch)
            def _():
                r0 = pl.multiple_of(ch * RCH, RCH)
                pltpu.sync_copy(table_hbm.at[pl.ds(r0, RCH)], tv)

                @pl.loop(0, RCH // 8)
                def _(g):
                    base = pl.multiple_of(g * 8, 8)
                    for rr in range(8):
                        sv = tv.at[base + rr]
                        dv = tv128.at[base + rr]
                        for k in range(d // 16):
                            dv[pl.ds(16 * k, 16)] = sv[pl.ds(16 * k, 16)]

                pltpu.sync_copy(tv128, padded_hbm.at[pl.ds(r0, RCH)])

    return run(table)


def _sc_gather(idx, padded, d):
    b, h = idx.shape
    spw = b // NW

    @functools.partial(
        pl.kernel,
        out_type=jax.ShapeDtypeStruct((b, h, d), jnp.float32),
        mesh=_mesh(),
        scratch_types=[
            pltpu.VMEM((NB, h), jnp.int32),
            pltpu.VMEM((NB, h, DP), jnp.float32),
            pltpu.VMEM((NB, h, d), jnp.float32),
            pltpu.SemaphoreType.DMA,
        ],
        compiler_params=pltpu.CompilerParams(use_tc_tiling_on_sc=True),
    )
    def run(idx_hbm, padded_hbm, out_hbm, idxv, rows, outv, sem):
        cid = lax.axis_index("c")
        sid = lax.axis_index("s")
        wid = sid * NC + cid

        @pl.loop(0, spw // NB)
        def _(i):
            b0 = wid * spw + i * NB
            pltpu.sync_copy(idx_hbm.at[pl.ds(b0, NB)], idxv)
            cs = []
            for j in range(NB):
                cs.append(
                    pltpu.async_copy(padded_hbm.at[idxv.at[j]], rows.at[j], sem)
                )
            for c0 in cs:
                c0.wait()

            @pl.loop(0, NB)
            def _(j):
                rv = rows.at[j]
                ov = outv.at[j]
                for r in range(h):
                    for k in range(d // 16):
                        ov[r, pl.ds(16 * k, 16)] = rv[r, pl.ds(16 * k, 16)]

            pltpu.sync_copy(outv, out_hbm.at[pl.ds(b0, NB)])

    return run(idx, padded)


def kernel(table, input):
    idx = input.astype(jnp.int32)
    padded = _repack(table)
    return _sc_gather(idx, padded, table.shape[1])


# speedup vs baseline: 1.1404x; 1.1404x over previous
import functools

import jax
import jax.numpy as jnp
from jax import lax
from jax.experimental import pallas as pl
from jax.experimental.pallas import tpu as pltpu
from jax.experimental.pallas import tpu_sc as plsc

NC, NS = 2, 16
NW = NC * NS
NB = 4
RCH = 200
DP = 128


def _mesh():
    return plsc.VectorSubcoreMesh(
        core_axis_name="c", subcore_axis_name="s", num_cores=NC, num_subcores=NS
    )


def _repack(table):
    v, d = table.shape
    nch = v // RCH

    @functools.partial(
        pl.kernel,
        out_type=jax.ShapeDtypeStruct((v, DP), jnp.float32),
        mesh=_mesh(),
        scratch_types=[
            pltpu.VMEM((2, RCH, d), jnp.float32),
            pltpu.VMEM((RCH, DP), jnp.float32),
            pltpu.SemaphoreType.DMA,
            pltpu.SemaphoreType.DMA,
        ],
    )
    def run(table_hbm, padded_hbm, tv, tv128, sem0, sem1):
        cid = lax.axis_index("c")
        sid = lax.axis_index("s")
        wid = sid * NC + cid
        sems = (sem0, sem1)

        def chunk_of(i):
            return wid + i * NW

        def fire(i, buf):
            ch = chunk_of(i)

            @pl.when(ch < nch)
            def _():
                r0 = pl.multiple_of(ch * RCH, RCH)
                pltpu.async_copy(
                    table_hbm.at[pl.ds(r0, RCH)], tv.at[buf], sems[buf]
                )

        def drain_widen_store(i, buf):
            ch = chunk_of(i)

            @pl.when(ch < nch)
            def _():
                r0 = pl.multiple_of(ch * RCH, RCH)
                pltpu.make_async_copy(
                    table_hbm.at[pl.ds(r0, RCH)], tv.at[buf], sems[buf]
                ).wait()

                @pl.loop(0, RCH // 8)
                def _(g):
                    base = pl.multiple_of(g * 8, 8)
                    for rr in range(8):
                        sv = tv.at[buf, base + rr]
                        dv = tv128.at[base + rr]
                        for k in range(d // 16):
                            dv[pl.ds(16 * k, 16)] = sv[pl.ds(16 * k, 16)]

                pltpu.sync_copy(tv128, padded_hbm.at[pl.ds(r0, RCH)])

        niter = (nch + NW - 1) // NW
        fire(0, 0)

        @pl.loop(0, (niter + 1) // 2)
        def _(t):
            a = 2 * t
            fire(a + 1, 1)
            drain_widen_store(a, 0)
            fire(a + 2, 0)
            drain_widen_store(a + 1, 1)

    return run(table)


def _sc_gather(idx, padded, d):
    b, h = idx.shape
    spw = b // NW
    nchunk = spw // NB

    @functools.partial(
        pl.kernel,
        out_type=jax.ShapeDtypeStruct((b, h, d), jnp.float32),
        mesh=_mesh(),
        scratch_types=[
            pltpu.VMEM((2, NB, h), jnp.int32),
            pltpu.VMEM((2, NB, h, DP), jnp.float32),
            pltpu.VMEM((NB, h, d), jnp.float32),
            pltpu.SemaphoreType.DMA,
            pltpu.SemaphoreType.DMA,
        ],
    )
    def run(idx_hbm, padded_hbm, out_hbm, idxv, rows, outv, sem0, sem1):
        cid = lax.axis_index("c")
        sid = lax.axis_index("s")
        wid = sid * NC + cid
        base = wid * spw
        sems = (sem0, sem1)

        def fire(i, buf):
            @pl.when(i < nchunk)
            def _():
                b0 = base + i * NB
                pltpu.sync_copy(idx_hbm.at[pl.ds(b0, NB)], idxv.at[buf])
                for j in range(NB):
                    pltpu.async_copy(
                        padded_hbm.at[idxv.at[buf, j]], rows.at[buf, j], sems[buf]
                    )

        def drain_compact_store(i, buf):
            @pl.when(i < nchunk)
            def _():
                b0 = base + i * NB
                for j in range(NB):
                    pltpu.make_async_copy(
                        padded_hbm.at[idxv.at[buf, j]], rows.at[buf, j], sems[buf]
                    ).wait()

                @pl.loop(0, NB)
                def _(j):
                    rv = rows.at[buf, j]
                    ov = outv.at[j]
                    for r in range(h):
                        for k in range(d // 16):
                            ov[r, pl.ds(16 * k, 16)] = rv[r, pl.ds(16 * k, 16)]

                pltpu.sync_copy(outv, out_hbm.at[pl.ds(b0, NB)])

        fire(0, 0)

        @pl.loop(0, nchunk // 2)
        def _(t):
            a = 2 * t
            fire(a + 1, 1)
            drain_compact_store(a, 0)
            fire(a + 2, 0)
            drain_compact_store(a + 1, 1)

    return run(idx, padded)


def kernel(table, input):
    idx = input.astype(jnp.int32)
    padded = _repack(table)
    return _sc_gather(idx, padded, table.shape[1])
